# baseline (speedup 1.0000x reference)
import functools

import jax
import jax.numpy as jnp
from jax import lax
from jax.experimental import pallas as pl
from jax.experimental.pallas import tpu as pltpu
from jax.experimental.pallas import tpu_sc as plsc

N = 8192
E = 8
D = 512
DH = D // 2
ROW = E + E * D

_info = plsc.get_sparse_core_info()
NC = _info.num_cores
NS = _info.num_subcores
L = _info.num_lanes
NW = NC * NS
TPT = N // NW
GT = 16
NG = TPT // GT
GSZ = GT * DH


def _sel_body(logits_ref, u_ref, off_ref):
    logits = logits_ref[...]
    u = u_ref[...]
    uc = jnp.clip(u, 1e-6, 1.0 - 1e-6)
    g = -jnp.log(-jnp.log(uc))
    s = logits + g
    m = jnp.max(s, axis=1, keepdims=True)
    lane = lax.broadcasted_iota(jnp.int32, s.shape, 1)
    sel = jnp.min(jnp.where(s == m, lane, E), axis=1, keepdims=True)
    n = (lax.broadcasted_iota(jnp.int32, (512, 1), 0)
         + 512 * pl.program_id(0))
    off_ref[...] = n * ROW + E + sel * D


def _selector_offsets(logits, u):
    return pl.pallas_call(
        _sel_body,
        grid=(N // 512,),
        in_specs=[
            pl.BlockSpec((512, E), lambda i: (i, 0)),
            pl.BlockSpec((512, E), lambda i: (i, 0)),
        ],
        out_specs=pl.BlockSpec((512, 1), lambda i: (i, 0)),
        out_shape=jax.ShapeDtypeStruct((N, 1), jnp.int32),
    )(logits, u)


@functools.partial(
    pl.kernel,
    mesh=plsc.VectorSubcoreMesh(core_axis_name="c", subcore_axis_name="s"),
    out_type=jax.ShapeDtypeStruct((N * DH,), jnp.float32),
    scratch_types=[
        pltpu.VMEM((TPT + L,), jnp.int32),
        pltpu.VMEM((4 * GT * D,), jnp.float32),
        pltpu.VMEM((2 * GSZ,), jnp.float32),
        pltpu.VMEM((2 * GSZ,), jnp.float32),
        pltpu.SemaphoreType.DMA,
        pltpu.SemaphoreType.DMA,
        pltpu.SemaphoreType.DMA,
        pltpu.SemaphoreType.DMA,
        pltpu.SemaphoreType.DMA,
        pltpu.SemaphoreType.DMA,
    ],
)
def _sc_gather(params_hbm, offs_hbm, eps_hbm, out_hbm,
               off_v, pbuf, ebuf, obuf,
               sem_p0, sem_p1, sem_e0, sem_e1, sem_o0, sem_o1):
    wid = lax.axis_index("s") * NC + lax.axis_index("c")
    base = wid * TPT

    pltpu.sync_copy(offs_hbm.at[pl.ds(base, TPT)], off_v.at[pl.ds(0, TPT)])

    def issue_param_group(g, sem):
        quarter = lax.rem(g, 4)

        def issue_one(tl, c):
            t = g * GT + tl
            off = pl.multiple_of(off_v[pl.ds(t, L)][0], 8)
            pltpu.make_async_copy(
                params_hbm.at[pl.ds(off, D)],
                pbuf.at[pl.ds((quarter * GT + tl) * D, D)], sem).start()
            return c

        lax.fori_loop(0, GT, issue_one, 0)

    def drain_param_group(sem):
        def wait_one(tl, c):
            pltpu.make_async_copy(
                params_hbm.at[pl.ds(0, D)],
                pbuf.at[pl.ds(0, D)], sem).wait()
            return c

        lax.fori_loop(0, GT, wait_one, 0)

    def issue_eps(g, sem):
        half = lax.rem(g, 2)
        pltpu.make_async_copy(
            eps_hbm.at[pl.ds((base + g * GT) * DH, GSZ)],
            ebuf.at[pl.ds(half * GSZ, GSZ)], sem).start()

    def wait_eps(sem):
        pltpu.make_async_copy(
            eps_hbm.at[pl.ds(0, GSZ)], ebuf.at[pl.ds(0, GSZ)], sem).wait()

    def issue_out(g, sem):
        half = lax.rem(g, 2)
        pltpu.make_async_copy(
            obuf.at[pl.ds(half * GSZ, GSZ)],
            out_hbm.at[pl.ds((base + g * GT) * DH, GSZ)], sem).start()

    def wait_out(sem):
        pltpu.make_async_copy(
            obuf.at[pl.ds(0, GSZ)],
            out_hbm.at[pl.ds(0, GSZ)], sem).wait()

    issue_param_group(0, sem_p0)
    issue_eps(0, sem_e0)
    issue_param_group(1, sem_p1)
    issue_eps(1, sem_e1)

    def even_half(g):
        drain_param_group(sem_p0)
        wait_eps(sem_e0)
        pl.when(g >= 2)(lambda: wait_out(sem_o0))

        def issue_next():
            issue_param_group(g + 2, sem_p0)
            issue_eps(g + 2, sem_e0)

        pl.when(g + 2 < NG)(issue_next)

    def odd_half(g):
        drain_param_group(sem_p1)
        wait_eps(sem_e1)
        pl.when(g >= 2)(lambda: wait_out(sem_o1))

        def issue_next():
            issue_param_group(g + 2, sem_p1)
            issue_eps(g + 2, sem_e1)

        pl.when(g + 2 < NG)(issue_next)

    def group_body(g, carry):
        parity = lax.rem(g, 2)
        pl.when(parity == 0)(lambda: even_half(g))
        pl.when(parity == 1)(lambda: odd_half(g))
        quarter = lax.rem(g, 4)
        half = parity

        def tok_body(tl, c):
            pslot = (quarter * GT + tl) * D
            eslot = half * GSZ + tl * DH
            for j in range(DH // L):
                mean = pbuf[pl.ds(pslot + j * L, L)]
                lstd = pbuf[pl.ds(pslot + DH + j * L, L)]
                ev = ebuf[pl.ds(eslot + j * L, L)]
                obuf[pl.ds(eslot + j * L, L)] = mean + ev * jnp.exp(lstd)
            return c

        lax.fori_loop(0, GT, tok_body, 0)
        pl.when(parity == 0)(lambda: issue_out(g, sem_o0))
        pl.when(parity == 1)(lambda: issue_out(g, sem_o1))
        return carry

    lax.fori_loop(0, NG, group_body, 0)
    wait_out(sem_o0)
    wait_out(sem_o1)


@jax.jit
def kernel(params, u, eps):
    logits = params[:, :E]
    offs = _selector_offsets(logits, u).reshape(N)
    out = _sc_gather(params.reshape(-1), offs, eps.reshape(-1))
    return out.reshape(N, DH)


# speedup vs baseline: 1.8156x; 1.8156x over previous
import jax
import jax.numpy as jnp
from jax import lax
from jax.experimental import pallas as pl

N = 8192
E = 8
D = 512
DH = D // 2
ROW = E + E * D
TB = 256


def _body(p_ref, u_ref, e_ref, o_ref):
    p = p_ref[...]
    u = u_ref[...]
    eps = e_ref[...]
    logits = p[:, :E]
    uc = jnp.clip(u, 1e-6, 1.0 - 1e-6)
    g = -jnp.log(-jnp.log(uc))
    s = logits + g
    m = jnp.max(s, axis=1, keepdims=True)
    lane = lax.broadcasted_iota(jnp.int32, s.shape, 1)
    sel = jnp.min(jnp.where(s == m, lane, E), axis=1, keepdims=True)
    mean = jnp.zeros((TB, DH), jnp.float32)
    lstd = jnp.zeros((TB, DH), jnp.float32)
    for e in range(E):
        msk = sel == e
        mean = jnp.where(msk, p[:, E + e * D:E + e * D + DH], mean)
        lstd = jnp.where(msk, p[:, E + e * D + DH:E + (e + 1) * D], lstd)
    o_ref[...] = mean + eps * jnp.exp(lstd)


@jax.jit
def kernel(params, u, eps):
    return pl.pallas_call(
        _body,
        grid=(N // TB,),
        in_specs=[
            pl.BlockSpec((TB, ROW), lambda i: (i, 0)),
            pl.BlockSpec((TB, E), lambda i: (i, 0)),
            pl.BlockSpec((TB, DH), lambda i: (i, 0)),
        ],
        out_specs=pl.BlockSpec((TB, DH), lambda i: (i, 0)),
        out_shape=jax.ShapeDtypeStruct((N, DH), jnp.float32),
    )(params, u, eps)


# speedup vs baseline: 1.8815x; 1.0363x over previous
import jax
import jax.numpy as jnp
from jax import lax
from jax.experimental import pallas as pl

N = 8192
E = 8
D = 512
DH = D // 2
ROW = E + E * D
TB = 512


def _body(p_ref, u_ref, e_ref, o_ref):
    p = p_ref[...]
    u = u_ref[...]
    eps = e_ref[...]
    logits = p[:, :E]
    uc = jnp.clip(u, 1e-6, 1.0 - 1e-6)
    g = -jnp.log(-jnp.log(uc))
    s = logits + g
    m = jnp.max(s, axis=1, keepdims=True)
    lane = lax.broadcasted_iota(jnp.int32, s.shape, 1)
    sel = jnp.min(jnp.where(s == m, lane, E), axis=1, keepdims=True)
    mean = jnp.zeros((TB, DH), jnp.float32)
    lstd = jnp.zeros((TB, DH), jnp.float32)
    for e in range(E):
        msk = sel == e
        mean = jnp.where(msk, p[:, E + e * D:E + e * D + DH], mean)
        lstd = jnp.where(msk, p[:, E + e * D + DH:E + (e + 1) * D], lstd)
    o_ref[...] = mean + eps * jnp.exp(lstd)


@jax.jit
def kernel(params, u, eps):
    return pl.pallas_call(
        _body,
        grid=(N // TB,),
        in_specs=[
            pl.BlockSpec((TB, ROW), lambda i: (i, 0)),
            pl.BlockSpec((TB, E), lambda i: (i, 0)),
            pl.BlockSpec((TB, DH), lambda i: (i, 0)),
        ],
        out_specs=pl.BlockSpec((TB, DH), lambda i: (i, 0)),
        out_shape=jax.ShapeDtypeStruct((N, DH), jnp.float32),
    )(params, u, eps)
